# baseline (speedup 1.0000x reference)
import functools

import jax
import jax.numpy as jnp
from jax import lax
from jax.experimental import pallas as pl
from jax.experimental.pallas import tpu as pltpu
from jax.experimental.pallas import tpu_sc as plsc

BATCH = 16384
EMBED = 32
HIDDEN = 256
_PACK = 8
_WPR = EMBED // 2

_NC, _NS = 2, 16
_NW = _NC * _NS
_B_PER_W = BATCH // _NW
_ICHUNK = 128
_NICHUNK = _B_PER_W // _ICHUNK
_OROWS = _B_PER_W // _PACK


def _sc_gather(user_id, movie_id, upk, mpk):
    mesh = plsc.VectorSubcoreMesh(core_axis_name="c", subcore_axis_name="s")

    @functools.partial(
        pl.kernel,
        mesh=mesh,
        out_type=[
            pltpu.HBM((_NW, _OROWS, 128), jnp.int32),
            pltpu.HBM((_NW, _OROWS, 128), jnp.int32),
        ],
        scratch_types=[
            pltpu.VMEM((_B_PER_W,), jnp.int32),
            pltpu.VMEM((_B_PER_W,), jnp.int32),
            pltpu.VMEM((_B_PER_W,), jnp.int32),
            pltpu.VMEM((_B_PER_W,), jnp.int32),
            pltpu.VMEM((2, _ICHUNK, 128), jnp.int32),
            pltpu.VMEM((2, _ICHUNK, 128), jnp.int32),
            pltpu.VMEM((_OROWS, 128), jnp.int32),
            pltpu.VMEM((_OROWS, 128), jnp.int32),
            pltpu.SemaphoreType.DMA,
        ],
        compiler_params=pltpu.CompilerParams(needs_layout_passes=False),
    )
    def k(uid_hbm, mid_hbm, utab_hbm, mtab_hbm, uout_hbm, mout_hbm,
          uidx_v, midx_v, ug_v, mg_v, ulines_v, mlines_v,
          uout_v, mout_v, sem):
        wid = lax.axis_index("s") * _NC + lax.axis_index("c")
        base = wid * _B_PER_W
        pltpu.sync_copy(uid_hbm.at[pl.ds(base, _B_PER_W)], uidx_v)
        pltpu.sync_copy(mid_hbm.at[pl.ds(base, _B_PER_W)], midx_v)
        for k16 in range(_B_PER_W // 16):
            sl = pl.ds(k16 * 16, 16)
            ug_v[sl] = lax.shift_right_logical(uidx_v[sl], 3)
            mg_v[sl] = lax.shift_right_logical(midx_v[sl], 3)

        lane = lax.iota(jnp.int32, 16)

        def fire(c):
            sl = pl.ds(c * _ICHUNK, _ICHUNK)
            buf = c % 2
            ucp = pltpu.async_copy(
                utab_hbm.at[ug_v.at[sl]], ulines_v.at[buf], sem)
            mcp = pltpu.async_copy(
                mtab_hbm.at[mg_v.at[sl]], mlines_v.at[buf], sem)
            return ucp, mcp

        def select_block(idx_v, lines_buf, out_v, c, k16):
            idx16 = idx_v[pl.ds(c * _ICHUNK + k16 * 16, 16)]
            i16 = lane + k16 * 16
            col_base = lax.bitwise_and(idx16, _PACK - 1) * _WPR
            out_base = (i16 + c * _ICHUNK) * _WPR
            for w in range(_WPR):
                vals = plsc.load_gather(lines_buf, [i16, col_base + w])
                pos = out_base + w
                plsc.store_scatter(
                    out_v,
                    [lax.shift_right_logical(pos, 7),
                     lax.bitwise_and(pos, 127)],
                    vals)

        cps = fire(0)
        for c in range(_NICHUNK):
            nxt = fire(c + 1) if c + 1 < _NICHUNK else None
            buf = c % 2
            cps[0].wait()

            def ubody(k16, _, c=c, buf=buf):
                select_block(uidx_v, ulines_v.at[buf], uout_v, c, k16)
                return _

            lax.fori_loop(0, _ICHUNK // 16, ubody, 0)
            cps[1].wait()

            def mbody(k16, _, c=c, buf=buf):
                select_block(midx_v, mlines_v.at[buf], mout_v, c, k16)
                return _

            lax.fori_loop(0, _ICHUNK // 16, mbody, 0)
            cps = nxt

        pltpu.sync_copy(uout_v, uout_hbm.at[wid])
        pltpu.sync_copy(mout_v, mout_hbm.at[wid])

    return k(user_id, movie_id, upk, mpk)


def _mlp_body(u8_ref, m8_ref, w1u_ref, w1m_ref, b1_ref, w2_ref, b2_ref,
              o_ref):
    u8 = u8_ref[...]
    m8 = m8_ref[...]
    outs = []
    for r in range(_PACK):
        sl = slice(r * EMBED, (r + 1) * EMBED)
        x = (jnp.dot(u8[:, sl], w1u_ref[...],
                     preferred_element_type=jnp.float32)
             + jnp.dot(m8[:, sl], w1m_ref[...],
                       preferred_element_type=jnp.float32)
             + b1_ref[...])
        h = jnp.maximum(x, 0.0)
        outs.append(jnp.dot(h, w2_ref[...],
                            preferred_element_type=jnp.float32))
    o_ref[...] = jnp.concatenate(outs, axis=1) + b2_ref[...]


def _tc_mlp(u8, m8, W1u, W1m, b1, W2, b2, block_m=512):
    grid = (BATCH // _PACK // block_m,)
    return pl.pallas_call(
        _mlp_body,
        grid=grid,
        in_specs=[
            pl.BlockSpec((block_m, _PACK * EMBED), lambda i: (i, 0)),
            pl.BlockSpec((block_m, _PACK * EMBED), lambda i: (i, 0)),
            pl.BlockSpec((EMBED, HIDDEN), lambda i: (0, 0)),
            pl.BlockSpec((EMBED, HIDDEN), lambda i: (0, 0)),
            pl.BlockSpec((1, HIDDEN), lambda i: (0, 0)),
            pl.BlockSpec((HIDDEN, 1), lambda i: (0, 0)),
            pl.BlockSpec((1, _PACK), lambda i: (0, 0)),
        ],
        out_specs=pl.BlockSpec((block_m, _PACK), lambda i: (i, 0)),
        out_shape=jax.ShapeDtypeStruct((BATCH // _PACK, _PACK), jnp.float32),
    )(u8, m8, W1u, W1m, b1, W2, b2)


def _pack_table(table):
    b = table.astype(jnp.bfloat16)
    return jax.lax.bitcast_convert_type(
        b.reshape(-1, 128, 2), jnp.int32)


def _unpack_rows(out3):
    b = jax.lax.bitcast_convert_type(
        out3.reshape(BATCH // _PACK, 128), jnp.bfloat16)
    return b.reshape(BATCH // _PACK, _PACK * EMBED)


def kernel(user_id, movie_title, user_table, movie_table, W1, b1, W2, b2):
    uid = user_id.astype(jnp.int32)
    mid = movie_title.astype(jnp.int32)
    upk = _pack_table(user_table)
    mpk = _pack_table(movie_table)
    uout, mout = _sc_gather(uid, mid, upk, mpk)
    u8 = _unpack_rows(uout)
    m8 = _unpack_rows(mout)
    W1u = W1[:EMBED].astype(jnp.bfloat16)
    W1m = W1[EMBED:].astype(jnp.bfloat16)
    b2x = jnp.broadcast_to(b2.reshape(1, 1), (1, _PACK))
    o8 = _tc_mlp(u8, m8, W1u, W1m, b1.reshape(1, HIDDEN), W2, b2x)
    return o8.reshape(BATCH, 1)


# speedup vs baseline: 15.0398x; 15.0398x over previous
import functools

import jax
import jax.numpy as jnp
from jax import lax
from jax.experimental import pallas as pl
from jax.experimental.pallas import tpu as pltpu
from jax.experimental.pallas import tpu_sc as plsc

BATCH = 16384
EMBED = 32
HIDDEN = 256

_NC, _NS = 2, 16
_NW = _NC * _NS
_B_PER_W = BATCH // _NW
_ICHUNK = 128
_NICHUNK = _B_PER_W // _ICHUNK


def _sc_gather(user_id, movie_id, utab128, mtab128):
    mesh = plsc.VectorSubcoreMesh(core_axis_name="c", subcore_axis_name="s")

    @functools.partial(
        pl.kernel,
        mesh=mesh,
        out_type=[
            pltpu.HBM((BATCH, 128), jnp.float32),
            pltpu.HBM((BATCH, 128), jnp.float32),
        ],
        scratch_types=[
            pltpu.VMEM((_B_PER_W,), jnp.int32),
            pltpu.VMEM((_B_PER_W,), jnp.int32),
            pltpu.VMEM((2, _ICHUNK, 128), jnp.float32),
            pltpu.VMEM((2, _ICHUNK, 128), jnp.float32),
            pltpu.SemaphoreType.DMA,
        ],
    )
    def k(uid_hbm, mid_hbm, utab_hbm, mtab_hbm, uout_hbm, mout_hbm,
          uidx_v, midx_v, ulines_v, mlines_v, sem):
        wid = lax.axis_index("s") * _NC + lax.axis_index("c")
        base = wid * _B_PER_W
        pltpu.sync_copy(uid_hbm.at[pl.ds(base, _B_PER_W)], uidx_v)
        pltpu.sync_copy(mid_hbm.at[pl.ds(base, _B_PER_W)], midx_v)

        def fire(c):
            sl = pl.ds(c * _ICHUNK, _ICHUNK)
            buf = c % 2
            ucp = pltpu.async_copy(
                utab_hbm.at[uidx_v.at[sl]], ulines_v.at[buf], sem)
            mcp = pltpu.async_copy(
                mtab_hbm.at[midx_v.at[sl]], mlines_v.at[buf], sem)
            return ucp, mcp

        cps = fire(0)
        for c in range(_NICHUNK):
            nxt = fire(c + 1) if c + 1 < _NICHUNK else None
            buf = c % 2
            out_sl = pl.ds(base + c * _ICHUNK, _ICHUNK)
            cps[0].wait()
            pltpu.sync_copy(ulines_v.at[buf], uout_hbm.at[out_sl])
            cps[1].wait()
            pltpu.sync_copy(mlines_v.at[buf], mout_hbm.at[out_sl])
            cps = nxt

    return k(user_id, movie_id, utab128, mtab128)


def _mlp_body(u_ref, m_ref, w1u_ref, w1m_ref, b1_ref, w2_ref, b2_ref, o_ref):
    x = (jnp.dot(u_ref[:, :EMBED], w1u_ref[...],
                 preferred_element_type=jnp.float32)
         + jnp.dot(m_ref[:, :EMBED], w1m_ref[...],
                   preferred_element_type=jnp.float32)
         + b1_ref[...])
    h = jnp.maximum(x, 0.0)
    o_ref[...] = (jnp.dot(h, w2_ref[...], preferred_element_type=jnp.float32)
                  + b2_ref[...])


def _tc_mlp(u128, m128, W1u, W1m, b1, W2, b2, block_m=2048):
    grid = (BATCH // block_m,)
    return pl.pallas_call(
        _mlp_body,
        grid=grid,
        in_specs=[
            pl.BlockSpec((block_m, 128), lambda i: (i, 0)),
            pl.BlockSpec((block_m, 128), lambda i: (i, 0)),
            pl.BlockSpec((EMBED, HIDDEN), lambda i: (0, 0)),
            pl.BlockSpec((EMBED, HIDDEN), lambda i: (0, 0)),
            pl.BlockSpec((1, HIDDEN), lambda i: (0, 0)),
            pl.BlockSpec((HIDDEN, 1), lambda i: (0, 0)),
            pl.BlockSpec((1, 1), lambda i: (0, 0)),
        ],
        out_specs=pl.BlockSpec((block_m, 1), lambda i: (i, 0)),
        out_shape=jax.ShapeDtypeStruct((BATCH, 1), jnp.float32),
    )(u128, m128, W1u, W1m, b1, W2, b2)


def kernel(user_id, movie_title, user_table, movie_table, W1, b1, W2, b2):
    uid = user_id.astype(jnp.int32)
    mid = movie_title.astype(jnp.int32)
    utab128 = jnp.pad(user_table, ((0, 0), (0, 128 - EMBED)))
    mtab128 = jnp.pad(movie_table, ((0, 0), (0, 128 - EMBED)))
    u128, m128 = _sc_gather(uid, mid, utab128, mtab128)
    W1u = W1[:EMBED]
    W1m = W1[EMBED:]
    return _tc_mlp(u128, m128, W1u, W1m,
                   b1.reshape(1, HIDDEN), W2, b2.reshape(1, 1))
